# baseline (speedup 1.0000x reference)
import functools

import jax
import jax.numpy as jnp
from jax import lax
from jax.experimental import pallas as pl
from jax.experimental.pallas import tpu as pltpu
from jax.experimental.pallas import tpu_sc as plsc

TIME_DIM = 64
NUM_TIME_BINS = 1000
MAX_TIME_DIFF = 26000000.0
BATCH = 16384
SEQ = 200

N = BATCH * SEQ
NW = 32
N_PER_W = N // NW
CHUNK = 1024
NCHUNKS = N_PER_W // CHUNK
GATHER_W = 128
NGATHER = CHUNK // GATHER_W
LANES = 16


def _sc_lookup(cur_hbm, nbr_hbm, table_hbm, out_hbm,
               cur_v, nbr_v, idx_v, rows_v, in_sem, g_sem):
    wid = lax.axis_index("s") * 2 + lax.axis_index("c")
    wbase = wid * N_PER_W

    def chunk_body(c, _):
        base = wbase + c * CHUNK
        cp_c = pltpu.async_copy(cur_hbm.at[pl.ds(base, CHUNK)], cur_v, in_sem)
        cp_n = pltpu.async_copy(nbr_hbm.at[pl.ds(base, CHUNK)], nbr_v, in_sem)
        cp_c.wait()
        cp_n.wait()

        for j in range(NGATHER):
            def disc_body(i2, carry):
                s = j * GATHER_W + i2 * LANES
                c16 = cur_v[pl.ds(s, LANES)]
                n16 = nbr_v[pl.ds(s, LANES)]
                d = c16 - n16
                cl = jnp.minimum(jnp.maximum(d, 0.0), MAX_TIME_DIFF)
                b = ((cl / MAX_TIME_DIFF) * NUM_TIME_BINS).astype(jnp.int32)
                idx_v[j, pl.ds(i2 * LANES, LANES)] = jnp.minimum(b, NUM_TIME_BINS)
                return carry
            lax.fori_loop(0, GATHER_W // LANES, disc_body, 0)

        cps = [
            pltpu.async_copy(table_hbm.at[idx_v.at[j]],
                             rows_v.at[pl.ds(j * GATHER_W, GATHER_W)], g_sem)
            for j in range(NGATHER)
        ]
        for cp in cps:
            cp.wait()

        pltpu.sync_copy(rows_v, out_hbm.at[pl.ds(base, CHUNK)])
        return _

    lax.fori_loop(0, NCHUNKS, chunk_body, 0)


def kernel(current_times, neighbor_times, lpe_weight):
    mesh = plsc.VectorSubcoreMesh(core_axis_name="c", subcore_axis_name="s")
    k = functools.partial(
        pl.kernel,
        out_type=jax.ShapeDtypeStruct((N, TIME_DIM), jnp.float32),
        mesh=mesh,
        scratch_types=[
            pltpu.VMEM((CHUNK,), jnp.float32),
            pltpu.VMEM((CHUNK,), jnp.float32),
            pltpu.VMEM((NGATHER, GATHER_W), jnp.int32),
            pltpu.VMEM((CHUNK, TIME_DIM), jnp.float32),
            pltpu.SemaphoreType.DMA,
            pltpu.SemaphoreType.DMA,
        ],
        compiler_params=pltpu.CompilerParams(use_tc_tiling_on_sc=False),
    )(_sc_lookup)
    out = k(current_times.reshape(N), neighbor_times.reshape(N), lpe_weight)
    return out.reshape(BATCH, SEQ, TIME_DIM)


# speedup vs baseline: 9.8851x; 9.8851x over previous
import functools

import jax
import jax.numpy as jnp
from jax import lax
from jax.experimental import pallas as pl
from jax.experimental.pallas import tpu as pltpu
from jax.experimental.pallas import tpu_sc as plsc

TIME_DIM = 64
NUM_TIME_BINS = 1000
MAX_TIME_DIFF = 26000000.0
BATCH = 16384
SEQ = 200

N = BATCH * SEQ
NW = 32
N_PER_W = N // NW
CHUNK = 320
NCHUNKS = N_PER_W // CHUNK
LANES = 16
VW = TIME_DIM // LANES
UNROLL = 8
TABLE_WORDS = (NUM_TIME_BINS + 1) * TIME_DIM


def _sc_lookup(cur_hbm, nbr_hbm, table_hbm, out_hbm,
               table_v, cur_v, nbr_v, offs_v, rows_v, in_sem, out_sem):
    wid = lax.axis_index("s") * 2 + lax.axis_index("c")
    wbase = wid * N_PER_W

    def issue_in(g, p):
        base = wbase + jnp.minimum(g, NCHUNKS - 1) * CHUNK
        pltpu.async_copy(cur_hbm.at[pl.ds(base, CHUNK)], cur_v.at[p], in_sem)
        pltpu.async_copy(nbr_hbm.at[pl.ds(base, CHUNK)], nbr_v.at[p], in_sem)

    def wait_in(p):
        pltpu.make_async_copy(cur_hbm.at[pl.ds(0, CHUNK)], cur_v.at[p], in_sem).wait()
        pltpu.make_async_copy(nbr_hbm.at[pl.ds(0, CHUNK)], nbr_v.at[p], in_sem).wait()

    def wait_out(p):
        pltpu.make_async_copy(rows_v.at[p], out_hbm.at[pl.ds(0, CHUNK * TIME_DIM)],
                              out_sem).wait()

    def discretize(p):
        def disc_body(i2, carry):
            s = i2 * LANES
            c16 = cur_v[p, pl.ds(s, LANES)]
            n16 = nbr_v[p, pl.ds(s, LANES)]
            d = c16 - n16
            cl = jnp.minimum(jnp.maximum(d, 0.0), MAX_TIME_DIFF)
            b = ((cl / MAX_TIME_DIFF) * NUM_TIME_BINS).astype(jnp.int32)
            b = jnp.minimum(b, NUM_TIME_BINS)
            offs_v[p, pl.ds(s, LANES)] = b * TIME_DIM
            return carry
        lax.fori_loop(0, CHUNK // LANES, disc_body, 0)

    def gather(p):
        def g_body(i, carry):
            offv = offs_v[p, pl.ds(i * LANES, LANES)]
            ebase = i * (LANES * TIME_DIM)
            for u in range(LANES):
                off = offv[u]
                for c in range(VW):
                    rows_v[p, pl.ds(ebase + u * TIME_DIM + c * LANES, LANES)] = (
                        table_v[pl.ds(off + c * LANES, LANES)])
            return carry
        lax.fori_loop(0, CHUNK // LANES, g_body, 0)

    def issue_out(g, p):
        base = (wbase + g * CHUNK) * TIME_DIM
        pltpu.async_copy(rows_v.at[p], out_hbm.at[pl.ds(base, CHUNK * TIME_DIM)],
                         out_sem)

    pltpu.sync_copy(table_hbm, table_v)

    issue_in(0, 0)
    for g in (0, 1):
        p = g % 2
        wait_in(p)
        issue_in(g + 1, 1 - p)
        discretize(p)
        gather(p)
        issue_out(g, p)

    def pair_body(k, carry):
        for sub in (0, 1):
            g = 2 * k + sub
            wait_in(sub)
            issue_in(g + 1, 1 - sub)
            discretize(sub)
            wait_out(sub)
            gather(sub)
            issue_out(g, sub)
        return carry
    lax.fori_loop(1, NCHUNKS // 2, pair_body, 0)

    wait_in(0)
    wait_out(0)
    wait_out(1)


def kernel(current_times, neighbor_times, lpe_weight):
    mesh = plsc.VectorSubcoreMesh(core_axis_name="c", subcore_axis_name="s")
    k = functools.partial(
        pl.kernel,
        out_type=jax.ShapeDtypeStruct((N * TIME_DIM,), jnp.float32),
        mesh=mesh,
        scratch_types=[
            pltpu.VMEM((TABLE_WORDS,), jnp.float32),
            pltpu.VMEM((2, CHUNK), jnp.float32),
            pltpu.VMEM((2, CHUNK), jnp.float32),
            pltpu.VMEM((2, CHUNK), jnp.int32),
            pltpu.VMEM((2, CHUNK * TIME_DIM), jnp.float32),
            pltpu.SemaphoreType.DMA,
            pltpu.SemaphoreType.DMA,
        ],
        compiler_params=pltpu.CompilerParams(use_tc_tiling_on_sc=False),
    )(_sc_lookup)
    out = k(current_times.reshape(N), neighbor_times.reshape(N),
            lpe_weight.reshape(TABLE_WORDS))
    return out.reshape(BATCH, SEQ, TIME_DIM)
